# Initial kernel scaffold; baseline (speedup 1.0000x reference)
import jax
import jax.numpy as jnp
from jax.experimental import pallas as pl


def kernel(x, v, W_s, W_v, W_pos, W_msg, W_rad, W_upd, b_upd, W_vdown, W_vgate, W_vup, W_sout, W_vout):
    raise NotImplementedError("write your pallas kernel here")


# speedup vs baseline: 123.5979x; 123.5979x over previous
import jax
import jax.numpy as jnp
from jax.experimental import pallas as pl

G = 128
NN = 24
BD = 512
HID = 256
MSG = 64
NR = 32
CUT = 2.0
L = 2
EPS = 0.1
OUTD = 6

GB = 16
CB = 1024

_F32 = jnp.float32


def _dot(a, b):
    return jax.lax.dot(a, b, preferred_element_type=_F32)


def _decoder_body(x_ref, v2_ref, Ws_ref, Wv_ref, xn_ref, vn_ref):
    xn_ref[...] = _dot(x_ref[...], Ws_ref[...])
    vn_ref[...] = _dot(v2_ref[...], Wv_ref[...])


def _mp_body(v_ref, xn_ref, vn_ref, Wpos_ref, Wmsg_ref, Wrad_ref, Wupd_ref,
             bupd_ref, Wvdown_ref, Wvgate_ref, Wvup_ref, Wsout_ref,
             Wvout_ref, out_ref):
    v2 = v_ref[...].reshape(GB * 3, BD)
    dirs = _dot(v2, Wpos_ref[...]).reshape(GB, 3, NN)
    nrm = jnp.sqrt(dirs[:, 0] ** 2 + dirs[:, 1] ** 2 + dirs[:, 2] ** 2)
    inv = 1.0 / (EPS + nrm)
    pos = [dirs[:, c] * inv for c in range(3)]
    diff = [p[:, None, :] - p[:, :, None] for p in pos]
    d = jnp.sqrt(diff[0] ** 2 + diff[1] ** 2 + diff[2] ** 2)
    inv_d = 1.0 / (d + 1e-8)
    dir_u = [df * inv_d for df in diff]
    u = d * (1.0 / CUT)
    u5 = u * u * u * u * u
    env = 1.0 - 21.0 * u5 + 35.0 * u5 * u - 15.0 * u5 * u * u
    env = jnp.where(u < 1.0, env, 0.0)
    ii = jax.lax.broadcasted_iota(jnp.int32, (NN, NN), 0)
    jj = jax.lax.broadcasted_iota(jnp.int32, (NN, NN), 1)
    env = jnp.where((ii != jj)[None], env, 0.0)
    spacing = CUT / (NR - 1)
    gamma = 1.0 / (2.0 * spacing * spacing)
    centers = jax.lax.broadcasted_iota(
        jnp.int32, (1, 1, 1, NR), 3).astype(_F32) * spacing
    rbf = jnp.exp(-gamma * (d[..., None] - centers) ** 2) * env[..., None]
    rbf2 = rbf.reshape(GB * NN * NN, NR)

    xn = xn_ref[...].reshape(GB * NN, HID)
    vn = [vn_ref[:, c].reshape(GB * NN, HID) for c in range(3)]
    for l in range(L):
        xs = _dot(xn, Wmsg_ref[l]).reshape(GB, NN, 1, MSG)
        R = _dot(rbf2, Wrad_ref[l]).reshape(GB, NN, NN, MSG)
        m = jax.nn.gelu(xs * R)
        agg = jnp.sum(m, axis=1).reshape(GB * NN, MSG)
        Wu = Wupd_ref[l]
        upd = (_dot(xn, Wu[:HID]) + _dot(agg, Wu[HID:])
               + bupd_ref[...][l:l + 1, :])
        xn = xn + jax.nn.gelu(upd)
        gate = _dot(rbf2, Wvgate_ref[l]).reshape(GB, NN, NN, MSG)
        for c in range(3):
            vproj = _dot(vn[c], Wvdown_ref[l]).reshape(GB, NN, 1, MSG)
            aggv = jnp.sum(vproj * gate + dir_u[c][..., None] * m,
                           axis=1).reshape(GB * NN, MSG)
            vn[c] = vn[c] + _dot(aggv, Wvup_ref[l])
    s_out = _dot(xn, Wsout_ref[...]).reshape(GB, NN, OUTD - 3)
    vouts = [_dot(vn[c], Wvout_ref[...]).reshape(GB, NN, 1) for c in range(3)]
    out_ref[...] = jnp.concatenate([s_out] + vouts, axis=-1)


def kernel(x, v, W_s, W_v, W_pos, W_msg, W_rad, W_upd, b_upd, W_vdown,
           W_vgate, W_vup, W_sout, W_vout):
    v2 = v.reshape(G * 3, BD)
    nblk = (HID * NN) // CB
    xn_wide, vn_wide = pl.pallas_call(
        _decoder_body,
        grid=(nblk,),
        in_specs=[
            pl.BlockSpec((G, BD), lambda j: (0, 0)),
            pl.BlockSpec((G * 3, BD), lambda j: (0, 0)),
            pl.BlockSpec((BD, CB), lambda j: (0, j)),
            pl.BlockSpec((BD, CB), lambda j: (0, j)),
        ],
        out_specs=[
            pl.BlockSpec((G, CB), lambda j: (0, j)),
            pl.BlockSpec((G * 3, CB), lambda j: (0, j)),
        ],
        out_shape=[
            jax.ShapeDtypeStruct((G, HID * NN), _F32),
            jax.ShapeDtypeStruct((G * 3, HID * NN), _F32),
        ],
    )(x, v2, W_s, W_v)
    xn = xn_wide.reshape(G, NN, HID)
    vn = vn_wide.reshape(G, 3, NN, HID)

    def full(shape):
        nd = len(shape)
        return pl.BlockSpec(shape, lambda i, _n=nd: (0,) * _n)

    out = pl.pallas_call(
        _mp_body,
        grid=(G // GB,),
        in_specs=[
            pl.BlockSpec((GB, 3, BD), lambda i: (i, 0, 0)),
            pl.BlockSpec((GB, NN, HID), lambda i: (i, 0, 0)),
            pl.BlockSpec((GB, 3, NN, HID), lambda i: (i, 0, 0, 0)),
            full((BD, NN)),
            full((L, HID, MSG)),
            full((L, NR, MSG)),
            full((L, HID + MSG, HID)),
            full((L, HID)),
            full((L, HID, MSG)),
            full((L, NR, MSG)),
            full((L, MSG, HID)),
            full((HID, OUTD - 3)),
            full((HID, 1)),
        ],
        out_specs=pl.BlockSpec((GB, NN, OUTD), lambda i: (i, 0, 0)),
        out_shape=jax.ShapeDtypeStruct((G, NN, OUTD), _F32),
    )(v, xn, vn, W_pos, W_msg, W_rad, W_upd, b_upd, W_vdown, W_vgate,
      W_vup, W_sout, W_vout)
    return out.reshape(G * NN, OUTD)
